# Initial kernel scaffold; baseline (speedup 1.0000x reference)
import jax
import jax.numpy as jnp
from jax.experimental import pallas as pl


def kernel(x, codebooks):
    raise NotImplementedError("write your pallas kernel here")


# speedup vs baseline: 2.0979x; 2.0979x over previous
import functools

import jax
import jax.numpy as jnp
from jax.experimental import pallas as pl
from jax.experimental.pallas import tpu as pltpu

NQ = 4
K = 512
D = 32
COMMIT = 0.25


def _rvq_kernel(x_ref, cb_ref, qsum_ref, qlist_ref, loss_ref, perp_ref,
                h1_scr, h2_scr, h3_scr, b2_scr, hist_scr, sse_scr,
                *, n_total, blk):
    i = pl.program_id(0)
    nblk = pl.num_programs(0)

    @pl.when(i == 0)
    def _init():
        cb = cb_ref[...]
        h1 = cb.astype(jnp.bfloat16)
        r1 = cb - h1.astype(jnp.float32)
        h2 = r1.astype(jnp.bfloat16)
        r2 = r1 - h2.astype(jnp.float32)
        h1_scr[...] = h1
        h2_scr[...] = h2
        h3_scr[...] = r2.astype(jnp.bfloat16)
        b2_scr[...] = jnp.sum(cb * cb, axis=-1)
        hist_scr[...] = jnp.zeros_like(hist_scr)
        for l in range(NQ):
            sse_scr[0, l] = 0.0

    r = x_ref[...]
    qsum = jnp.zeros_like(r)
    iota_k = jax.lax.broadcasted_iota(jnp.int32, (blk, K), 1)
    for l in range(NQ):
        m = jax.lax.dot_general(
            r.astype(jnp.bfloat16), h1_scr[l],
            (((1,), (1,)), ((), ())),
            preferred_element_type=jnp.float32,
        )
        a2 = jnp.sum(r * r, axis=1, keepdims=True)
        dist = (a2 + b2_scr[l:l + 1, :]) - 2.0 * m
        mn = jnp.min(dist, axis=1, keepdims=True)
        code = jnp.min(jnp.where(dist <= mn, iota_k, K), axis=1)
        onehot = (iota_k == code[:, None]).astype(jnp.float32)
        ob = onehot.astype(jnp.bfloat16)
        dims = (((1,), (0,)), ((), ()))
        q = (jax.lax.dot_general(ob, h1_scr[l], dims,
                                 preferred_element_type=jnp.float32)
             + jax.lax.dot_general(ob, h2_scr[l], dims,
                                   preferred_element_type=jnp.float32)) \
            + jax.lax.dot_general(ob, h3_scr[l], dims,
                                  preferred_element_type=jnp.float32)
        hist_scr[l:l + 1, :] += jnp.sum(onehot, axis=0)[None, :]
        t = q - r
        sse_scr[0, l] += jnp.sum(t * t)
        st = r + t
        qlist_ref[l] = st
        qsum = qsum + st
        r = r - st
    qsum_ref[...] = qsum

    @pl.when(i == nblk - 1)
    def _finish():
        probs = hist_scr[...] * (1.0 / n_total)
        ent = -jnp.sum(probs * jnp.log(probs + 1e-10), axis=1)
        perp_ref[...] = jnp.exp(ent)[None, :]
        sse_total = sse_scr[0, 0] + sse_scr[0, 1] + sse_scr[0, 2] + sse_scr[0, 3]
        loss_ref[...] = jnp.full((1, 1), sse_total * (COMMIT / (NQ * n_total * D)),
                                 dtype=jnp.float32)


def kernel(x, codebooks):
    orig_shape = x.shape
    n = x.size // D
    flat = x.reshape(n, D)
    if n % 2048 == 0:
        blk = 2048
    elif n % 512 == 0:
        blk = 512
    else:
        blk = n
    nblk = n // blk

    body = functools.partial(_rvq_kernel, n_total=n, blk=blk)
    qsum, qlist, loss, perp = pl.pallas_call(
        body,
        grid=(nblk,),
        in_specs=[
            pl.BlockSpec((blk, D), lambda i: (i, 0)),
            pl.BlockSpec((NQ, K, D), lambda i: (0, 0, 0)),
        ],
        out_specs=[
            pl.BlockSpec((blk, D), lambda i: (i, 0)),
            pl.BlockSpec((NQ, blk, D), lambda i: (0, i, 0)),
            pl.BlockSpec((1, 1), lambda i: (0, 0)),
            pl.BlockSpec((1, NQ), lambda i: (0, 0)),
        ],
        out_shape=[
            jax.ShapeDtypeStruct((n, D), jnp.float32),
            jax.ShapeDtypeStruct((NQ, n, D), jnp.float32),
            jax.ShapeDtypeStruct((1, 1), jnp.float32),
            jax.ShapeDtypeStruct((1, NQ), jnp.float32),
        ],
        scratch_shapes=[
            pltpu.VMEM((NQ, K, D), jnp.bfloat16),
            pltpu.VMEM((NQ, K, D), jnp.bfloat16),
            pltpu.VMEM((NQ, K, D), jnp.bfloat16),
            pltpu.VMEM((NQ, K), jnp.float32),
            pltpu.VMEM((NQ, K), jnp.float32),
            pltpu.SMEM((1, NQ), jnp.float32),
        ],
    )(flat, codebooks)

    return (
        qsum.reshape(orig_shape),
        loss.reshape(()),
        qlist.reshape((NQ,) + orig_shape),
        perp.reshape(NQ),
    )


# speedup vs baseline: 3.4968x; 1.6668x over previous
import functools

import jax
import jax.numpy as jnp
from jax.experimental import pallas as pl
from jax.experimental.pallas import tpu as pltpu

NQ = 4
K = 512
D = 32
COMMIT = 0.25


def _rvq_kernel(x_ref, cb_ref, qsum_ref, qlist_ref, loss_ref, perp_ref,
                cb1_scr, g1_scr, g2_scr, g3_scr, b2_scr, hacc_scr, sse_scr,
                *, n_total, blk):
    i = pl.program_id(0)
    nblk = pl.num_programs(0)

    @pl.when(i == 0)
    def _init():
        cb = cb_ref[...]
        h1 = cb.astype(jnp.bfloat16)
        r1 = cb - h1.astype(jnp.float32)
        h2 = r1.astype(jnp.bfloat16)
        r2 = r1 - h2.astype(jnp.float32)
        h3 = r2.astype(jnp.bfloat16)
        cb1_scr[...] = h1
        g1_scr[...] = jnp.swapaxes(h1, 1, 2)
        g2_scr[...] = jnp.swapaxes(h2, 1, 2)
        g3_scr[...] = jnp.swapaxes(h3, 1, 2)
        b2_scr[...] = jnp.sum(cb * cb, axis=-1, keepdims=True)
        hacc_scr[...] = jnp.zeros_like(hacc_scr)
        for l in range(NQ):
            sse_scr[0, l] = 0.0

    rT = jnp.transpose(x_ref[...])
    qsumT = jnp.zeros_like(rT)
    siota = jax.lax.broadcasted_iota(jnp.int32, (K, blk), 0)
    for l in range(NQ):
        mT = jax.lax.dot_general(
            cb1_scr[l], rT.astype(jnp.bfloat16),
            (((1,), (0,)), ((), ())),
            preferred_element_type=jnp.float32,
        )
        dist = b2_scr[l] - 2.0 * mT
        mn = jnp.min(dist, axis=0, keepdims=True)
        codeT = jnp.min(jnp.where(dist <= mn, siota, K), axis=0, keepdims=True)
        oT = (siota == codeT).astype(jnp.bfloat16)
        dims = (((1,), (0,)), ((), ()))
        qT = (jax.lax.dot_general(g1_scr[l], oT, dims,
                                  preferred_element_type=jnp.float32)
              + jax.lax.dot_general(g2_scr[l], oT, dims,
                                    preferred_element_type=jnp.float32)) \
            + jax.lax.dot_general(g3_scr[l], oT, dims,
                                  preferred_element_type=jnp.float32)
        hacc_scr[l] += oT
        tT = qT - rT
        sse_scr[0, l] += jnp.sum(tT * tT)
        stT = rT + tT
        qlist_ref[l] = jnp.transpose(stT)
        qsumT = qsumT + stT
        rT = rT - stT
    qsum_ref[...] = jnp.transpose(qsumT)

    @pl.when(i == nblk - 1)
    def _finish():
        hist = jnp.sum(hacc_scr[...].astype(jnp.float32), axis=2)
        probs = hist * (1.0 / n_total)
        ent = -jnp.sum(probs * jnp.log(probs + 1e-10), axis=1)
        perp_ref[...] = jnp.exp(ent)[None, :]
        sse_total = sse_scr[0, 0] + sse_scr[0, 1] + sse_scr[0, 2] + sse_scr[0, 3]
        loss_ref[...] = jnp.full((1, 1), sse_total * (COMMIT / (NQ * n_total * D)),
                                 dtype=jnp.float32)


def kernel(x, codebooks):
    orig_shape = x.shape
    n = x.size // D
    flat = x.reshape(n, D)
    if n % 2048 == 0:
        blk = 2048
    elif n % 512 == 0:
        blk = 512
    else:
        blk = n
    nblk = n // blk

    body = functools.partial(_rvq_kernel, n_total=n, blk=blk)
    qsum, qlist, loss, perp = pl.pallas_call(
        body,
        grid=(nblk,),
        in_specs=[
            pl.BlockSpec((blk, D), lambda i: (i, 0)),
            pl.BlockSpec((NQ, K, D), lambda i: (0, 0, 0)),
        ],
        out_specs=[
            pl.BlockSpec((blk, D), lambda i: (i, 0)),
            pl.BlockSpec((NQ, blk, D), lambda i: (0, i, 0)),
            pl.BlockSpec((1, 1), lambda i: (0, 0)),
            pl.BlockSpec((1, NQ), lambda i: (0, 0)),
        ],
        out_shape=[
            jax.ShapeDtypeStruct((n, D), jnp.float32),
            jax.ShapeDtypeStruct((NQ, n, D), jnp.float32),
            jax.ShapeDtypeStruct((1, 1), jnp.float32),
            jax.ShapeDtypeStruct((1, NQ), jnp.float32),
        ],
        scratch_shapes=[
            pltpu.VMEM((NQ, K, D), jnp.bfloat16),
            pltpu.VMEM((NQ, D, K), jnp.bfloat16),
            pltpu.VMEM((NQ, D, K), jnp.bfloat16),
            pltpu.VMEM((NQ, D, K), jnp.bfloat16),
            pltpu.VMEM((NQ, K, 1), jnp.float32),
            pltpu.VMEM((NQ, K, blk), jnp.bfloat16),
            pltpu.SMEM((1, NQ), jnp.float32),
        ],
    )(flat, codebooks)

    return (
        qsum.reshape(orig_shape),
        loss.reshape(()),
        qlist.reshape((NQ,) + orig_shape),
        perp.reshape(NQ),
    )


# speedup vs baseline: 4.5026x; 1.2876x over previous
import functools

import jax
import jax.numpy as jnp
from jax.experimental import pallas as pl
from jax.experimental.pallas import tpu as pltpu

NQ = 4
K = 512
D = 32
DA = 48
COMMIT = 0.25
BIG = 1e12


def _rvq_kernel(x_ref, cb_ref, qsum_ref, qlist_ref, loss_ref, perp_ref,
                cba_scr, g_scr, raug_scr, hacc_scr, sse_scr,
                *, n_total, blk):
    i = pl.program_id(0)
    nblk = pl.num_programs(0)

    @pl.when(i == 0)
    def _init():
        cb = cb_ref[...]
        h1 = cb.astype(jnp.bfloat16)
        r1 = cb - h1.astype(jnp.float32)
        h2 = r1.astype(jnp.bfloat16)
        r2 = r1 - h2.astype(jnp.float32)
        h3 = r2.astype(jnp.bfloat16)
        nb2 = -0.5 * jnp.sum(cb * cb, axis=-1, keepdims=True)
        s1 = nb2.astype(jnp.bfloat16)
        t1 = nb2 - s1.astype(jnp.float32)
        s2 = t1.astype(jnp.bfloat16)
        s3 = (t1 - s2.astype(jnp.float32)).astype(jnp.bfloat16)
        zpad = jnp.zeros((NQ, K, DA - D - 3), dtype=jnp.bfloat16)
        cba_scr[...] = jnp.concatenate([h1, s1, s2, s3, zpad], axis=2)
        g_scr[...] = jnp.concatenate(
            [jnp.swapaxes(h1, 1, 2), jnp.swapaxes(h2, 1, 2),
             jnp.swapaxes(h3, 1, 2)], axis=1)
        ones = jnp.ones((3, blk), dtype=jnp.bfloat16)
        zer = jnp.zeros((DA - D - 3, blk), dtype=jnp.bfloat16)
        raug_scr[D:, :] = jnp.concatenate([ones, zer], axis=0)
        hacc_scr[...] = jnp.zeros_like(hacc_scr)
        for l in range(NQ):
            sse_scr[0, l] = 0.0

    rT = jnp.transpose(x_ref[...])
    qsumT = jnp.zeros_like(rT)
    siota = jax.lax.broadcasted_iota(jnp.int32, (K, blk), 0).astype(jnp.float32)
    for l in range(NQ):
        raug_scr[:D, :] = rT.astype(jnp.bfloat16)
        s = jax.lax.dot_general(
            cba_scr[l], raug_scr[...],
            (((1,), (0,)), ((), ())),
            preferred_element_type=jnp.float32,
        )
        mx = jnp.max(s, axis=0, keepdims=True)
        t = (mx - s) * BIG + siota
        codeT = jnp.min(t, axis=0, keepdims=True)
        oT = (t == codeT).astype(jnp.bfloat16)
        q3 = jax.lax.dot_general(
            g_scr[l], oT, (((1,), (0,)), ((), ())),
            preferred_element_type=jnp.float32,
        )
        qT = (q3[:D] + q3[D:2 * D]) + q3[2 * D:]
        hacc_scr[l] += oT
        tT = qT - rT
        sse_scr[0, l] += jnp.sum(tT * tT)
        stT = rT + tT
        qlist_ref[l] = jnp.transpose(stT)
        qsumT = qsumT + stT
        rT = rT - stT
    qsum_ref[...] = jnp.transpose(qsumT)

    @pl.when(i == nblk - 1)
    def _finish():
        hist = jnp.sum(hacc_scr[...].astype(jnp.float32), axis=2)
        probs = hist * (1.0 / n_total)
        ent = -jnp.sum(probs * jnp.log(probs + 1e-10), axis=1)
        perp_ref[...] = jnp.exp(ent)[None, :]
        sse_total = sse_scr[0, 0] + sse_scr[0, 1] + sse_scr[0, 2] + sse_scr[0, 3]
        loss_ref[...] = jnp.full((1, 1), sse_total * (COMMIT / (NQ * n_total * D)),
                                 dtype=jnp.float32)


def kernel(x, codebooks):
    orig_shape = x.shape
    n = x.size // D
    flat = x.reshape(n, D)
    if n % 2048 == 0:
        blk = 2048
    elif n % 512 == 0:
        blk = 512
    else:
        blk = n
    nblk = n // blk

    body = functools.partial(_rvq_kernel, n_total=n, blk=blk)
    qsum, qlist, loss, perp = pl.pallas_call(
        body,
        grid=(nblk,),
        in_specs=[
            pl.BlockSpec((blk, D), lambda i: (i, 0)),
            pl.BlockSpec((NQ, K, D), lambda i: (0, 0, 0)),
        ],
        out_specs=[
            pl.BlockSpec((blk, D), lambda i: (i, 0)),
            pl.BlockSpec((NQ, blk, D), lambda i: (0, i, 0)),
            pl.BlockSpec((1, 1), lambda i: (0, 0)),
            pl.BlockSpec((1, NQ), lambda i: (0, 0)),
        ],
        out_shape=[
            jax.ShapeDtypeStruct((n, D), jnp.float32),
            jax.ShapeDtypeStruct((NQ, n, D), jnp.float32),
            jax.ShapeDtypeStruct((1, 1), jnp.float32),
            jax.ShapeDtypeStruct((1, NQ), jnp.float32),
        ],
        scratch_shapes=[
            pltpu.VMEM((NQ, K, DA), jnp.bfloat16),
            pltpu.VMEM((NQ, 3 * D, K), jnp.bfloat16),
            pltpu.VMEM((DA, blk), jnp.bfloat16),
            pltpu.VMEM((NQ, K, blk), jnp.bfloat16),
            pltpu.SMEM((1, NQ), jnp.float32),
        ],
    )(flat, codebooks)

    return (
        qsum.reshape(orig_shape),
        loss.reshape(()),
        qlist.reshape((NQ,) + orig_shape),
        perp.reshape(NQ),
    )


# speedup vs baseline: 4.5108x; 1.0018x over previous
import functools

import jax
import jax.numpy as jnp
from jax.experimental import pallas as pl
from jax.experimental.pallas import tpu as pltpu

NQ = 4
K = 512
D = 32
DA = 48
COMMIT = 0.25
BIG = 1e12


def _rvq_kernel(x_ref, cb_ref, qsum_ref, qlist_ref, hist_ref, svec_ref,
                cba_scr, g_scr, raug_scr, hacc_scr,
                *, blk):
    i = pl.program_id(1)
    ninner = pl.num_programs(1)

    @pl.when(i == 0)
    def _init():
        cb = cb_ref[...]
        h1 = cb.astype(jnp.bfloat16)
        r1 = cb - h1.astype(jnp.float32)
        h2 = r1.astype(jnp.bfloat16)
        r2 = r1 - h2.astype(jnp.float32)
        h3 = r2.astype(jnp.bfloat16)
        nb2 = -0.5 * jnp.sum(cb * cb, axis=-1, keepdims=True)
        s1 = nb2.astype(jnp.bfloat16)
        t1 = nb2 - s1.astype(jnp.float32)
        s2 = t1.astype(jnp.bfloat16)
        s3 = (t1 - s2.astype(jnp.float32)).astype(jnp.bfloat16)
        zpad = jnp.zeros((NQ, K, DA - D - 3), dtype=jnp.bfloat16)
        cba_scr[...] = jnp.concatenate([h1, s1, s2, s3, zpad], axis=2)
        g_scr[...] = jnp.concatenate(
            [jnp.swapaxes(h1, 1, 2), jnp.swapaxes(h2, 1, 2),
             jnp.swapaxes(h3, 1, 2)], axis=1)
        ones = jnp.ones((3, blk), dtype=jnp.bfloat16)
        zer = jnp.zeros((DA - D - 3, blk), dtype=jnp.bfloat16)
        raug_scr[D:, :] = jnp.concatenate([ones, zer], axis=0)
        hacc_scr[...] = jnp.zeros_like(hacc_scr)
        svec_ref[...] = jnp.zeros_like(svec_ref)

    rT = jnp.transpose(x_ref[...])
    qsumT = jnp.zeros_like(rT)
    siota = jax.lax.broadcasted_iota(jnp.int32, (K, blk), 0).astype(jnp.float32)
    for l in range(NQ):
        raug_scr[:D, :] = rT.astype(jnp.bfloat16)
        s = jax.lax.dot_general(
            cba_scr[l], raug_scr[...],
            (((1,), (0,)), ((), ())),
            preferred_element_type=jnp.float32,
        )
        mx = jnp.max(s, axis=0, keepdims=True)
        t = (mx - s) * BIG + siota
        codeT = jnp.min(t, axis=0, keepdims=True)
        oT = (t == codeT).astype(jnp.bfloat16)
        q3 = jax.lax.dot_general(
            g_scr[l], oT, (((1,), (0,)), ((), ())),
            preferred_element_type=jnp.float32,
        )
        qT = (q3[:D] + q3[D:2 * D]) + q3[2 * D:]
        hacc_scr[l] += oT
        tT = qT - rT
        svec_ref[0, l:l + 1, :] += jnp.sum(tT * tT, axis=0, keepdims=True)
        stT = rT + tT
        qlist_ref[l] = jnp.transpose(stT)
        qsumT = qsumT + stT
        rT = rT - stT
    qsum_ref[...] = jnp.transpose(qsumT)

    @pl.when(i == ninner - 1)
    def _finish():
        hist_ref[0] = jnp.sum(hacc_scr[...].astype(jnp.float32), axis=2)


def _finalize_kernel(hist_ref, svec_ref, loss_ref, perp_ref, *, n_total):
    hist = jnp.sum(hist_ref[...], axis=0)
    probs = hist * (1.0 / n_total)
    ent = -jnp.sum(probs * jnp.log(probs + 1e-10), axis=1)
    perp_ref[...] = jnp.exp(ent)[None, :]
    sse_total = jnp.sum(svec_ref[...])
    loss_ref[...] = jnp.full((1, 1), sse_total * (COMMIT / (NQ * n_total * D)),
                             dtype=jnp.float32)


def kernel(x, codebooks):
    orig_shape = x.shape
    n = x.size // D
    flat = x.reshape(n, D)
    if n % 2048 == 0:
        blk = 2048
    elif n % 512 == 0:
        blk = 512
    else:
        blk = n
    nblk = n // blk
    par = 2 if nblk % 2 == 0 else 1
    half = nblk // par

    body = functools.partial(_rvq_kernel, blk=blk)
    qsum, qlist, hist2, svec = pl.pallas_call(
        body,
        grid=(par, half),
        in_specs=[
            pl.BlockSpec((blk, D), lambda o, i: (o * half + i, 0)),
            pl.BlockSpec((NQ, K, D), lambda o, i: (0, 0, 0)),
        ],
        out_specs=[
            pl.BlockSpec((blk, D), lambda o, i: (o * half + i, 0)),
            pl.BlockSpec((NQ, blk, D), lambda o, i: (0, o * half + i, 0)),
            pl.BlockSpec((1, NQ, K), lambda o, i: (o, 0, 0)),
            pl.BlockSpec((1, NQ, blk), lambda o, i: (o, 0, 0)),
        ],
        out_shape=[
            jax.ShapeDtypeStruct((n, D), jnp.float32),
            jax.ShapeDtypeStruct((NQ, n, D), jnp.float32),
            jax.ShapeDtypeStruct((par, NQ, K), jnp.float32),
            jax.ShapeDtypeStruct((par, NQ, blk), jnp.float32),
        ],
        scratch_shapes=[
            pltpu.VMEM((NQ, K, DA), jnp.bfloat16),
            pltpu.VMEM((NQ, 3 * D, K), jnp.bfloat16),
            pltpu.VMEM((DA, blk), jnp.bfloat16),
            pltpu.VMEM((NQ, K, blk), jnp.bfloat16),
        ],
        compiler_params=pltpu.CompilerParams(
            dimension_semantics=("parallel", "arbitrary"),
        ),
    )(flat, codebooks)

    loss, perp = pl.pallas_call(
        functools.partial(_finalize_kernel, n_total=n),
        out_shape=[
            jax.ShapeDtypeStruct((1, 1), jnp.float32),
            jax.ShapeDtypeStruct((1, NQ), jnp.float32),
        ],
    )(hist2, svec)

    return (
        qsum.reshape(orig_shape),
        loss.reshape(()),
        qlist.reshape((NQ,) + orig_shape),
        perp.reshape(NQ),
    )


# speedup vs baseline: 4.6539x; 1.0317x over previous
import functools

import jax
import jax.numpy as jnp
from jax.experimental import pallas as pl
from jax.experimental.pallas import tpu as pltpu

NQ = 4
K = 512
D = 32
DA = 48
COMMIT = 0.25
BIG = 1e12


def _rvq_kernel(x_ref, cb_ref, qsum_ref, qlist_ref, loss_ref, perp_ref,
                cba_scr, g_scr, raug_scr, hacc_scr, sse_scr,
                *, n_total, blk):
    i = pl.program_id(0)
    nblk = pl.num_programs(0)

    @pl.when(i == 0)
    def _init():
        cb = cb_ref[...]
        h1 = cb.astype(jnp.bfloat16)
        r1 = cb - h1.astype(jnp.float32)
        h2 = r1.astype(jnp.bfloat16)
        r2 = r1 - h2.astype(jnp.float32)
        h3 = r2.astype(jnp.bfloat16)
        nb2 = -0.5 * jnp.sum(cb * cb, axis=-1, keepdims=True)
        s1 = nb2.astype(jnp.bfloat16)
        t1 = nb2 - s1.astype(jnp.float32)
        s2 = t1.astype(jnp.bfloat16)
        s3 = (t1 - s2.astype(jnp.float32)).astype(jnp.bfloat16)
        zpad = jnp.zeros((NQ, K, DA - D - 3), dtype=jnp.bfloat16)
        cba_scr[...] = jnp.concatenate([h1, s1, s2, s3, zpad], axis=2)
        g_scr[...] = jnp.concatenate(
            [jnp.swapaxes(h1, 1, 2), jnp.swapaxes(h2, 1, 2),
             jnp.swapaxes(h3, 1, 2)], axis=1)
        ones = jnp.ones((3, blk), dtype=jnp.bfloat16)
        zer = jnp.zeros((DA - D - 3, blk), dtype=jnp.bfloat16)
        raug_scr[D:, :] = jnp.concatenate([ones, zer], axis=0)
        hacc_scr[...] = jnp.zeros_like(hacc_scr)
        for l in range(NQ):
            sse_scr[0, l] = 0.0

    rT = jnp.transpose(x_ref[...])
    qsumT = jnp.zeros_like(rT)
    siota = jax.lax.broadcasted_iota(jnp.int32, (K, blk), 0).astype(jnp.float32)
    for l in range(NQ):
        raug_scr[:D, :] = rT.astype(jnp.bfloat16)
        s = jax.lax.dot_general(
            cba_scr[l], raug_scr[...],
            (((1,), (0,)), ((), ())),
            preferred_element_type=jnp.float32,
        )
        mx = jnp.max(s, axis=0, keepdims=True)
        codeT = jnp.min((mx - s) * BIG + siota, axis=0, keepdims=True)
        oT = (siota == codeT).astype(jnp.bfloat16)
        q3 = jax.lax.dot_general(
            g_scr[l], oT, (((1,), (0,)), ((), ())),
            preferred_element_type=jnp.float32,
        )
        qT = (q3[:D] + q3[D:2 * D]) + q3[2 * D:]
        hacc_scr[l] += oT
        tT = qT - rT
        sse_scr[0, l] += jnp.sum(tT * tT)
        stT = rT + tT
        qlist_ref[l] = jnp.transpose(stT)
        qsumT = qsumT + stT
        rT = rT - stT
    qsum_ref[...] = jnp.transpose(qsumT)

    @pl.when(i == nblk - 1)
    def _finish():
        hist = jnp.sum(hacc_scr[...].astype(jnp.float32), axis=2)
        probs = hist * (1.0 / n_total)
        ent = -jnp.sum(probs * jnp.log(probs + 1e-10), axis=1)
        perp_ref[...] = jnp.exp(ent)[None, :]
        sse_total = sse_scr[0, 0] + sse_scr[0, 1] + sse_scr[0, 2] + sse_scr[0, 3]
        loss_ref[...] = jnp.full((1, 1), sse_total * (COMMIT / (NQ * n_total * D)),
                                 dtype=jnp.float32)


def kernel(x, codebooks):
    orig_shape = x.shape
    n = x.size // D
    flat = x.reshape(n, D)
    if n % 2048 == 0:
        blk = 2048
    elif n % 512 == 0:
        blk = 512
    else:
        blk = n
    nblk = n // blk

    body = functools.partial(_rvq_kernel, n_total=n, blk=blk)
    qsum, qlist, loss, perp = pl.pallas_call(
        body,
        grid=(nblk,),
        in_specs=[
            pl.BlockSpec((blk, D), lambda i: (i, 0)),
            pl.BlockSpec((NQ, K, D), lambda i: (0, 0, 0)),
        ],
        out_specs=[
            pl.BlockSpec((blk, D), lambda i: (i, 0)),
            pl.BlockSpec((NQ, blk, D), lambda i: (0, i, 0)),
            pl.BlockSpec((1, 1), lambda i: (0, 0)),
            pl.BlockSpec((1, NQ), lambda i: (0, 0)),
        ],
        out_shape=[
            jax.ShapeDtypeStruct((n, D), jnp.float32),
            jax.ShapeDtypeStruct((NQ, n, D), jnp.float32),
            jax.ShapeDtypeStruct((1, 1), jnp.float32),
            jax.ShapeDtypeStruct((1, NQ), jnp.float32),
        ],
        scratch_shapes=[
            pltpu.VMEM((NQ, K, DA), jnp.bfloat16),
            pltpu.VMEM((NQ, 3 * D, K), jnp.bfloat16),
            pltpu.VMEM((DA, blk), jnp.bfloat16),
            pltpu.VMEM((NQ, K, blk), jnp.bfloat16),
            pltpu.SMEM((1, NQ), jnp.float32),
        ],
    )(flat, codebooks)

    return (
        qsum.reshape(orig_shape),
        loss.reshape(()),
        qlist.reshape((NQ,) + orig_shape),
        perp.reshape(NQ),
    )


# speedup vs baseline: 4.8180x; 1.0353x over previous
import functools

import jax
import jax.numpy as jnp
from jax.experimental import pallas as pl
from jax.experimental.pallas import tpu as pltpu

NQ = 4
K = 512
D = 32
DA = 48
COMMIT = 0.25
BIG = 1e12


def _rvq_kernel(x_ref, cb_ref, qsum_ref, qlist_ref, loss_ref, perp_ref,
                cba_scr, g_scr, raug_scr, hacc_scr, sse_scr,
                *, n_total, blk):
    i = pl.program_id(0)
    nblk = pl.num_programs(0)

    @pl.when(i == 0)
    def _init():
        cb = cb_ref[...]
        h1 = cb.astype(jnp.bfloat16)
        r1 = cb - h1.astype(jnp.float32)
        h2 = r1.astype(jnp.bfloat16)
        r2 = r1 - h2.astype(jnp.float32)
        h3 = r2.astype(jnp.bfloat16)
        nb2 = -0.5 * jnp.sum(cb * cb, axis=-1, keepdims=True)
        s1 = nb2.astype(jnp.bfloat16)
        t1 = nb2 - s1.astype(jnp.float32)
        s2 = t1.astype(jnp.bfloat16)
        s3 = (t1 - s2.astype(jnp.float32)).astype(jnp.bfloat16)
        zpad = jnp.zeros((NQ, K, DA - D - 3), dtype=jnp.bfloat16)
        cba_scr[...] = jnp.concatenate([h1, s1, s2, s3, zpad], axis=2)
        g_scr[...] = jnp.concatenate(
            [jnp.swapaxes(h1, 1, 2), jnp.swapaxes(h2, 1, 2),
             jnp.swapaxes(h3, 1, 2)], axis=1)
        ones = jnp.ones((3, blk), dtype=jnp.bfloat16)
        zer = jnp.zeros((DA - D - 3, blk), dtype=jnp.bfloat16)
        raug_scr[D:, :] = jnp.concatenate([ones, zer], axis=0)
        hacc_scr[...] = jnp.zeros_like(hacc_scr)
        for l in range(NQ):
            sse_scr[0, l] = 0.0

    rT = jnp.transpose(x_ref[...])
    qsumT = jnp.zeros_like(rT)
    siota = jax.lax.broadcasted_iota(jnp.int32, (K, blk), 0).astype(jnp.float32)
    for l in range(NQ):
        raug_scr[:D, :] = rT.astype(jnp.bfloat16)
        s = jax.lax.dot_general(
            cba_scr[l], raug_scr[...],
            (((1,), (0,)), ((), ())),
            preferred_element_type=jnp.float32,
        )
        mx = jnp.max(s, axis=0, keepdims=True)
        codeT = jnp.min((mx - s) * BIG + siota, axis=0, keepdims=True)
        oT = (siota == codeT).astype(jnp.bfloat16)
        q3 = jax.lax.dot_general(
            g_scr[l], oT, (((1,), (0,)), ((), ())),
            preferred_element_type=jnp.float32,
        )
        qT = (q3[:D] + q3[D:2 * D]) + q3[2 * D:]
        hacc_scr[l] += oT
        tT = qT - rT
        sse_scr[0, l] += jnp.sum(tT * tT)
        stT = rT + tT
        qlist_ref[l] = jnp.transpose(stT)
        qsumT = qsumT + stT
        rT = rT - stT
    qsum_ref[...] = jnp.transpose(qsumT)

    @pl.when(i == nblk - 1)
    def _finish():
        hist = jnp.sum(hacc_scr[...].astype(jnp.float32), axis=2)
        probs = hist * (1.0 / n_total)
        ent = -jnp.sum(probs * jnp.log(probs + 1e-10), axis=1)
        perp_ref[...] = jnp.exp(ent)[None, :]
        sse_total = sse_scr[0, 0] + sse_scr[0, 1] + sse_scr[0, 2] + sse_scr[0, 3]
        loss_ref[...] = jnp.full((1, 1), sse_total * (COMMIT / (NQ * n_total * D)),
                                 dtype=jnp.float32)


def kernel(x, codebooks):
    orig_shape = x.shape
    n = x.size // D
    flat = x.reshape(n, D)
    if n % 4096 == 0:
        blk = 4096
    elif n % 512 == 0:
        blk = 512
    else:
        blk = n
    nblk = n // blk

    body = functools.partial(_rvq_kernel, n_total=n, blk=blk)
    qsum, qlist, loss, perp = pl.pallas_call(
        body,
        grid=(nblk,),
        in_specs=[
            pl.BlockSpec((blk, D), lambda i: (i, 0)),
            pl.BlockSpec((NQ, K, D), lambda i: (0, 0, 0)),
        ],
        out_specs=[
            pl.BlockSpec((blk, D), lambda i: (i, 0)),
            pl.BlockSpec((NQ, blk, D), lambda i: (0, i, 0)),
            pl.BlockSpec((1, 1), lambda i: (0, 0)),
            pl.BlockSpec((1, NQ), lambda i: (0, 0)),
        ],
        out_shape=[
            jax.ShapeDtypeStruct((n, D), jnp.float32),
            jax.ShapeDtypeStruct((NQ, n, D), jnp.float32),
            jax.ShapeDtypeStruct((1, 1), jnp.float32),
            jax.ShapeDtypeStruct((1, NQ), jnp.float32),
        ],
        scratch_shapes=[
            pltpu.VMEM((NQ, K, DA), jnp.bfloat16),
            pltpu.VMEM((NQ, 3 * D, K), jnp.bfloat16),
            pltpu.VMEM((DA, blk), jnp.bfloat16),
            pltpu.VMEM((NQ, K, blk), jnp.bfloat16),
            pltpu.SMEM((1, NQ), jnp.float32),
        ],
    )(flat, codebooks)

    return (
        qsum.reshape(orig_shape),
        loss.reshape(()),
        qlist.reshape((NQ,) + orig_shape),
        perp.reshape(NQ),
    )
